# Initial kernel scaffold; baseline (speedup 1.0000x reference)
import functools

import jax
import jax.numpy as jnp
from jax import lax
from jax.experimental import pallas as pl
from jax.experimental.pallas import tpu as pltpu, tpu_sc as plsc

NUM_CHAR = 12000
NUM_WORD = 100000
EMB = 50
B = 1024
L = 256
N = B * L
N2 = 2 * N

_info = plsc.get_sparse_core_info()
NC, NS, LANES = _info.num_cores, _info.num_subcores, _info.num_lanes
NW = NC * NS
T = N // NW
C = 1024
CHUNKS = T // C
G = 128
K = (2 * C) // G


def _sc_body(comb_hbm, idxc_hbm, idxw_hbm, out_hbm, idxc_v, idxw_v, idx2_v,
             rows_v, sem):
    wid = lax.axis_index("s") * NC + lax.axis_index("c")
    it = lax.iota(jnp.int32, LANES)

    for k in range(CHUNKS):
        base = wid * T + k * C
        pltpu.sync_copy(idxc_hbm.at[pl.ds(base, C)], idxc_v)
        pltpu.sync_copy(idxw_hbm.at[pl.ds(base, C)], idxw_v)

        def build(j, carry):
            vc = idxc_v[pl.ds(j * LANES, LANES)]
            vw = idxw_v[pl.ds(j * LANES, LANES)] + NUM_CHAR
            pos = 2 * LANES * j + 2 * it
            row = lax.shift_right_logical(pos, 7)
            col = lax.bitwise_and(pos, G - 1)
            plsc.store_scatter(idx2_v, [row, col], vc)
            plsc.store_scatter(idx2_v, [row, col + 1], vw)
            return carry

        lax.fori_loop(0, C // LANES, build, 0)

        copies = [
            pltpu.async_copy(comb_hbm.at[idx2_v.at[g]],
                             rows_v.at[pl.ds(g * G, G)], sem)
            for g in range(K)
        ]
        for cp in copies:
            cp.wait()

        pltpu.sync_copy(rows_v, out_hbm.at[pl.ds(2 * base, 2 * C)])


@functools.partial(jax.jit, static_argnames=())
def _run(comb, idxc, idxw):
    kfn = pl.kernel(
        _sc_body,
        mesh=plsc.VectorSubcoreMesh(core_axis_name="c", subcore_axis_name="s"),
        out_type=jax.ShapeDtypeStruct((N2, EMB), jnp.float32),
        scratch_types=[
            pltpu.VMEM((C,), jnp.int32),
            pltpu.VMEM((C,), jnp.int32),
            pltpu.VMEM((K, G), jnp.int32),
            pltpu.VMEM((2 * C, EMB), jnp.float32),
            pltpu.SemaphoreType.DMA,
        ],
    )
    return kfn(comb, idxc, idxw)


def kernel(seqs_char, seqs_word, att_mask, char_table, word_table):
    del att_mask
    comb = jnp.concatenate([char_table, word_table], axis=0)
    idxc = seqs_char.reshape(-1).astype(jnp.int32)
    idxw = seqs_word.reshape(-1).astype(jnp.int32)
    out = _run(comb, idxc, idxw)
    return out.reshape(B, L, 2 * EMB)


# speedup vs baseline: 2.6535x; 2.6535x over previous
import functools

import jax
import jax.numpy as jnp
from jax import lax
from jax.experimental import pallas as pl
from jax.experimental.pallas import tpu as pltpu, tpu_sc as plsc

NUM_CHAR = 12000
NUM_WORD = 100000
EMB = 50
EMBP = 64
B = 1024
L = 256
N = B * L
N2 = 2 * N

_info = plsc.get_sparse_core_info()
NC, NS, LANES = _info.num_cores, _info.num_subcores, _info.num_lanes
NW = NC * NS
T = N // NW
C = 512
CHUNKS = T // C
G = 128
K = (2 * C) // G


def _sc_body(comb_hbm, idxc_hbm, idxw_hbm, out_hbm, idxc_v, idxw_v, idx2_v,
             rows_v, sem):
    wid = lax.axis_index("s") * NC + lax.axis_index("c")
    it = lax.iota(jnp.int32, LANES)

    for k in range(CHUNKS):
        base = wid * T + k * C
        pltpu.sync_copy(idxc_hbm.at[pl.ds(base, C)], idxc_v)
        pltpu.sync_copy(idxw_hbm.at[pl.ds(base, C)], idxw_v)

        def build(j, carry):
            vc = idxc_v[pl.ds(j * LANES, LANES)]
            vw = idxw_v[pl.ds(j * LANES, LANES)] + NUM_CHAR
            pos = 2 * LANES * j + 2 * it
            plsc.store_scatter(idx2_v, [pos], vc)
            plsc.store_scatter(idx2_v, [pos + 1], vw)
            return carry

        lax.fori_loop(0, C // LANES, build, 0)

        copies = [
            pltpu.async_copy(comb_hbm.at[idx2_v.at[pl.ds(g * G, G)]],
                             rows_v.at[pl.ds(g * G, G)], sem)
            for g in range(K)
        ]
        for cp in copies:
            cp.wait()

        pltpu.sync_copy(rows_v, out_hbm.at[pl.ds(2 * base, 2 * C)])


@jax.jit
def _run(comb, idxc, idxw):
    kfn = pl.kernel(
        _sc_body,
        mesh=plsc.VectorSubcoreMesh(core_axis_name="c", subcore_axis_name="s"),
        out_type=jax.ShapeDtypeStruct((N2, EMBP), jnp.float32),
        compiler_params=pltpu.CompilerParams(needs_layout_passes=False,
                                             use_tc_tiling_on_sc=False),
        scratch_types=[
            pltpu.VMEM((C,), jnp.int32),
            pltpu.VMEM((C,), jnp.int32),
            pltpu.VMEM((2 * C,), jnp.int32),
            pltpu.VMEM((2 * C, EMBP), jnp.float32),
            pltpu.SemaphoreType.DMA,
        ],
    )
    return kfn(comb, idxc, idxw)


def kernel(seqs_char, seqs_word, att_mask, char_table, word_table):
    del att_mask
    comb = jnp.concatenate([
        jnp.pad(char_table, ((0, 0), (0, EMBP - EMB))),
        jnp.pad(word_table, ((0, 0), (0, EMBP - EMB))),
    ], axis=0)
    idxc = seqs_char.reshape(-1).astype(jnp.int32)
    idxw = seqs_word.reshape(-1).astype(jnp.int32)
    out = _run(comb, idxc, idxw)
    return out[:, :EMB].reshape(B, L, 2 * EMB)


# speedup vs baseline: 4.7947x; 1.8069x over previous
import jax
import jax.numpy as jnp
from jax import lax
from jax.experimental import pallas as pl
from jax.experimental.pallas import tpu as pltpu, tpu_sc as plsc

NUM_CHAR = 12000
NUM_WORD = 100000
EMB = 50
EMBP = 64
B = 1024
L = 256
N = B * L
N2 = 2 * N

_info = plsc.get_sparse_core_info()
NC, NS, LANES = _info.num_cores, _info.num_subcores, _info.num_lanes
NW = NC * NS
T = N // NW
C = 256
CHUNKS = T // C
G = 128
K = (2 * C) // G


def _sc_body(comb_hbm, idxc_hbm, idxw_hbm, out_hbm, idxc_v, idxw_v, idx2_v,
             rows_v, pack_v, sem):
    wid = lax.axis_index("s") * NC + lax.axis_index("c")
    it = lax.iota(jnp.int32, LANES)
    m2 = it < 2

    for k in range(CHUNKS):
        base = wid * T + k * C
        pltpu.sync_copy(idxc_hbm.at[pl.ds(base, C)], idxc_v)
        pltpu.sync_copy(idxw_hbm.at[pl.ds(base, C)], idxw_v)

        def build(j, carry):
            vc = idxc_v[pl.ds(j * LANES, LANES)]
            vw = idxw_v[pl.ds(j * LANES, LANES)] + NUM_CHAR
            pos = 2 * LANES * j + 2 * it
            plsc.store_scatter(idx2_v, [pos], vc)
            plsc.store_scatter(idx2_v, [pos + 1], vw)
            return carry

        lax.fori_loop(0, C // LANES, build, 0)

        copies = [
            pltpu.async_copy(comb_hbm.at[idx2_v.at[pl.ds(g * G, G)]],
                             rows_v.at[pl.ds(g * G, G)], sem)
            for g in range(K)
        ]
        for cp in copies:
            cp.wait()

        def compose(t, carry):
            for jj in range(3):
                pack_v[t, pl.ds(jj * LANES, LANES)] = (
                    rows_v[2 * t, pl.ds(jj * LANES, LANES)])
            tail_c = rows_v[2 * t, pl.ds(48, LANES)]
            plsc.store_compressed(pack_v.at[t, pl.ds(48, LANES)], tail_c,
                                  mask=m2)
            for jj in range(3):
                pack_v[t, pl.ds(EMB + jj * LANES, LANES)] = (
                    rows_v[2 * t + 1, pl.ds(jj * LANES, LANES)])
            tail_w = rows_v[2 * t + 1, pl.ds(48, LANES)]
            plsc.store_compressed(pack_v.at[t, pl.ds(EMB + 48, LANES)],
                                  tail_w, mask=m2)
            return carry

        lax.fori_loop(0, C, compose, 0)

        pltpu.sync_copy(pack_v, out_hbm.at[pl.ds(base, C)])


@jax.jit
def _run(comb, idxc, idxw):
    kfn = pl.kernel(
        _sc_body,
        mesh=plsc.VectorSubcoreMesh(core_axis_name="c", subcore_axis_name="s"),
        out_type=jax.ShapeDtypeStruct((N, 128), jnp.float32),
        compiler_params=pltpu.CompilerParams(needs_layout_passes=False,
                                             use_tc_tiling_on_sc=False),
        scratch_types=[
            pltpu.VMEM((C,), jnp.int32),
            pltpu.VMEM((C,), jnp.int32),
            pltpu.VMEM((2 * C,), jnp.int32),
            pltpu.VMEM((2 * C, EMBP), jnp.float32),
            pltpu.VMEM((C, 128), jnp.float32),
            pltpu.SemaphoreType.DMA,
        ],
    )
    return kfn(comb, idxc, idxw)


def kernel(seqs_char, seqs_word, att_mask, char_table, word_table):
    del att_mask
    comb = jnp.concatenate([
        jnp.pad(char_table, ((0, 0), (0, EMBP - EMB))),
        jnp.pad(word_table, ((0, 0), (0, EMBP - EMB))),
    ], axis=0)
    idxc = seqs_char.reshape(-1).astype(jnp.int32)
    idxw = seqs_word.reshape(-1).astype(jnp.int32)
    out = _run(comb, idxc, idxw)
    return out[:, :2 * EMB].reshape(B, L, 2 * EMB)


# speedup vs baseline: 6.1378x; 1.2801x over previous
import jax
import jax.numpy as jnp
from jax import lax
from jax.experimental import pallas as pl
from jax.experimental.pallas import tpu as pltpu, tpu_sc as plsc

NUM_CHAR = 12000
NUM_WORD = 100000
EMB = 50
EMBP = 64
B = 1024
L = 256
N = B * L
N2 = 2 * N

_info = plsc.get_sparse_core_info()
NC, NS, LANES = _info.num_cores, _info.num_subcores, _info.num_lanes
NW = NC * NS
T = N // NW
C = 256
CHUNKS = T // C
G = 128
K = (2 * C) // G


def _sc_body(comb_hbm, idxc_hbm, idxw_hbm, out_hbm, idxc_v, idxw_v,
             idx2_a, idx2_b, rows_a, rows_b, pack_v, sem_a, sem_b):
    wid = lax.axis_index("s") * NC + lax.axis_index("c")
    it = lax.iota(jnp.int32, LANES)
    m2 = it < 2
    tbase = wid * T

    pltpu.sync_copy(idxc_hbm.at[pl.ds(tbase, T)], idxc_v)
    pltpu.sync_copy(idxw_hbm.at[pl.ds(tbase, T)], idxw_v)

    def build(kk, idx2_v):
        def bd(j, carry):
            vc = idxc_v[pl.ds(kk * C + j * LANES, LANES)]
            vw = idxw_v[pl.ds(kk * C + j * LANES, LANES)] + NUM_CHAR
            pos = 2 * LANES * j + 2 * it
            plsc.store_scatter(idx2_v, [pos], vc)
            plsc.store_scatter(idx2_v, [pos + 1], vw)
            return carry
        lax.fori_loop(0, C // LANES, bd, 0)

    def fire(idx2_v, rows_v, sem):
        for g in range(K):
            pltpu.async_copy(comb_hbm.at[idx2_v.at[pl.ds(g * G, G)]],
                             rows_v.at[pl.ds(g * G, G)], sem)

    def drain(rows_v, sem):
        for g in range(K):
            pltpu.make_async_copy(comb_hbm.at[idx2_a.at[pl.ds(0, G)]],
                                  rows_v.at[pl.ds(g * G, G)], sem).wait()

    def one_token(t, rows_v):
        for jj in range(3):
            pack_v[t, pl.ds(jj * LANES, LANES)] = (
                rows_v[2 * t, pl.ds(jj * LANES, LANES)])
        tail_c = rows_v[2 * t, pl.ds(48, LANES)]
        plsc.store_compressed(pack_v.at[t, pl.ds(48, LANES)], tail_c, mask=m2)
        for jj in range(3):
            pack_v[t, pl.ds(EMB + jj * LANES, LANES)] = (
                rows_v[2 * t + 1, pl.ds(jj * LANES, LANES)])
        tail_w = rows_v[2 * t + 1, pl.ds(48, LANES)]
        plsc.store_compressed(pack_v.at[t, pl.ds(EMB + 48, LANES)], tail_w,
                              mask=m2)

    def compose_and_write(kk, rows_v):
        def cp(t2, carry):
            one_token(2 * t2, rows_v)
            one_token(2 * t2 + 1, rows_v)
            return carry
        lax.fori_loop(0, C // 2, cp, 0)
        pltpu.sync_copy(pack_v, out_hbm.at[pl.ds(tbase + kk * C, C)])

    build(0, idx2_a)
    fire(idx2_a, rows_a, sem_a)

    def step(i, carry):
        kk = 2 * i
        build(kk + 1, idx2_b)
        fire(idx2_b, rows_b, sem_b)
        drain(rows_a, sem_a)
        compose_and_write(kk, rows_a)
        build(kk + 2, idx2_a)
        fire(idx2_a, rows_a, sem_a)
        drain(rows_b, sem_b)
        compose_and_write(kk + 1, rows_b)
        return carry

    lax.fori_loop(0, CHUNKS // 2 - 1, step, 0)

    kk = CHUNKS - 2
    build(kk + 1, idx2_b)
    fire(idx2_b, rows_b, sem_b)
    drain(rows_a, sem_a)
    compose_and_write(kk, rows_a)
    drain(rows_b, sem_b)
    compose_and_write(kk + 1, rows_b)


@jax.jit
def _run(comb, idxc, idxw):
    kfn = pl.kernel(
        _sc_body,
        mesh=plsc.VectorSubcoreMesh(core_axis_name="c", subcore_axis_name="s"),
        out_type=jax.ShapeDtypeStruct((N, 128), jnp.float32),
        compiler_params=pltpu.CompilerParams(needs_layout_passes=False,
                                             use_tc_tiling_on_sc=False),
        scratch_types=[
            pltpu.VMEM((T,), jnp.int32),
            pltpu.VMEM((T,), jnp.int32),
            pltpu.VMEM((2 * C,), jnp.int32),
            pltpu.VMEM((2 * C,), jnp.int32),
            pltpu.VMEM((2 * C, EMBP), jnp.float32),
            pltpu.VMEM((2 * C, EMBP), jnp.float32),
            pltpu.VMEM((C, 128), jnp.float32),
            pltpu.SemaphoreType.DMA,
            pltpu.SemaphoreType.DMA,
        ],
    )
    return kfn(comb, idxc, idxw)


def kernel(seqs_char, seqs_word, att_mask, char_table, word_table):
    del att_mask
    comb = jnp.concatenate([
        jnp.pad(char_table, ((0, 0), (0, EMBP - EMB))),
        jnp.pad(word_table, ((0, 0), (0, EMBP - EMB))),
    ], axis=0)
    idxc = seqs_char.reshape(-1).astype(jnp.int32)
    idxw = seqs_word.reshape(-1).astype(jnp.int32)
    out = _run(comb, idxc, idxw)
    return out[:, :2 * EMB].reshape(B, L, 2 * EMB)


# speedup vs baseline: 6.1386x; 1.0001x over previous
import jax
import jax.numpy as jnp
from jax import lax
from jax.experimental import pallas as pl
from jax.experimental.pallas import tpu as pltpu, tpu_sc as plsc

NUM_CHAR = 12000
NUM_WORD = 100000
EMB = 50
EMBP = 64
B = 1024
L = 256
N = B * L
N2 = 2 * N

_info = plsc.get_sparse_core_info()
NC, NS, LANES = _info.num_cores, _info.num_subcores, _info.num_lanes
NW = NC * NS
T = N // NW
C = 256
CHUNKS = T // C
G = 128
K = (2 * C) // G


def _sc_body(comb_hbm, idxc_hbm, idxw_hbm, out_hbm, idxc_v, idxw_v,
             idx2_a, idx2_b, rows_a, rows_b, pack_v, sem_a, sem_b):
    wid = lax.axis_index("s") * NC + lax.axis_index("c")
    it = lax.iota(jnp.int32, LANES)
    m2 = it < 2
    tbase = wid * T

    pltpu.sync_copy(idxc_hbm.at[pl.ds(tbase, T)], idxc_v)
    pltpu.sync_copy(idxw_hbm.at[pl.ds(tbase, T)], idxw_v)

    def build(kk, idx2_v):
        def bd(j, carry):
            vc = idxc_v[pl.ds(kk * C + j * LANES, LANES)]
            vw = idxw_v[pl.ds(kk * C + j * LANES, LANES)] + NUM_CHAR
            pos = 2 * LANES * j + 2 * it
            plsc.store_scatter(idx2_v, [pos], vc)
            plsc.store_scatter(idx2_v, [pos + 1], vw)
            return carry
        lax.fori_loop(0, C // LANES, bd, 0)

    def fire(idx2_v, rows_v, sem):
        for g in range(K):
            pltpu.async_copy(comb_hbm.at[idx2_v.at[pl.ds(g * G, G)]],
                             rows_v.at[pl.ds(g * G, G)], sem)

    def drain(rows_v, sem):
        for g in range(K):
            pltpu.make_async_copy(comb_hbm.at[idx2_a.at[pl.ds(0, G)]],
                                  rows_v.at[pl.ds(g * G, G)], sem).wait()

    def one_token(t, rows_v):
        for jj in range(3):
            pack_v[t, pl.ds(jj * LANES, LANES)] = (
                rows_v[2 * t, pl.ds(jj * LANES, LANES)])
        tail_c = rows_v[2 * t, pl.ds(48, LANES)]
        plsc.store_compressed(pack_v.at[t, pl.ds(48, LANES)], tail_c, mask=m2)
        for jj in range(3):
            pack_v[t, pl.ds(EMB + jj * LANES, LANES)] = (
                rows_v[2 * t + 1, pl.ds(jj * LANES, LANES)])
        tail_w = rows_v[2 * t + 1, pl.ds(48, LANES)]
        plsc.store_compressed(pack_v.at[t, pl.ds(EMB + 48, LANES)], tail_w,
                              mask=m2)

    def compose_and_write(kk, rows_v):
        def cp(t4, carry):
            for u in range(4):
                one_token(4 * t4 + u, rows_v)
            return carry
        lax.fori_loop(0, C // 4, cp, 0)
        pltpu.sync_copy(pack_v, out_hbm.at[pl.ds(tbase + kk * C, C)])

    build(0, idx2_a)
    fire(idx2_a, rows_a, sem_a)

    def step(i, carry):
        kk = 2 * i
        build(kk + 1, idx2_b)
        fire(idx2_b, rows_b, sem_b)
        drain(rows_a, sem_a)
        compose_and_write(kk, rows_a)
        build(kk + 2, idx2_a)
        fire(idx2_a, rows_a, sem_a)
        drain(rows_b, sem_b)
        compose_and_write(kk + 1, rows_b)
        return carry

    lax.fori_loop(0, CHUNKS // 2 - 1, step, 0)

    kk = CHUNKS - 2
    build(kk + 1, idx2_b)
    fire(idx2_b, rows_b, sem_b)
    drain(rows_a, sem_a)
    compose_and_write(kk, rows_a)
    drain(rows_b, sem_b)
    compose_and_write(kk + 1, rows_b)


@jax.jit
def _run(comb, idxc, idxw):
    kfn = pl.kernel(
        _sc_body,
        mesh=plsc.VectorSubcoreMesh(core_axis_name="c", subcore_axis_name="s"),
        out_type=jax.ShapeDtypeStruct((N, 128), jnp.float32),
        compiler_params=pltpu.CompilerParams(needs_layout_passes=False,
                                             use_tc_tiling_on_sc=False),
        scratch_types=[
            pltpu.VMEM((T,), jnp.int32),
            pltpu.VMEM((T,), jnp.int32),
            pltpu.VMEM((2 * C,), jnp.int32),
            pltpu.VMEM((2 * C,), jnp.int32),
            pltpu.VMEM((2 * C, EMBP), jnp.float32),
            pltpu.VMEM((2 * C, EMBP), jnp.float32),
            pltpu.VMEM((C, 128), jnp.float32),
            pltpu.SemaphoreType.DMA,
            pltpu.SemaphoreType.DMA,
        ],
    )
    return kfn(comb, idxc, idxw)


def kernel(seqs_char, seqs_word, att_mask, char_table, word_table):
    del att_mask
    comb = jnp.concatenate([
        jnp.pad(char_table, ((0, 0), (0, EMBP - EMB))),
        jnp.pad(word_table, ((0, 0), (0, EMBP - EMB))),
    ], axis=0)
    idxc = seqs_char.reshape(-1).astype(jnp.int32)
    idxw = seqs_word.reshape(-1).astype(jnp.int32)
    out = _run(comb, idxc, idxw)
    return out[:, :2 * EMB].reshape(B, L, 2 * EMB)
